# baseline (speedup 1.0000x reference)
import functools

import jax
import jax.numpy as jnp
from jax import lax
from jax.experimental import pallas as pl
from jax.experimental.pallas import tpu as pltpu
from jax.experimental.pallas import tpu_sc as plsc

_N_NODES = 10000
_N_HEDGES = 20000
_N_INC = 320000
_D = 128
_DP = _D // 2

_NS = 16
_PER_TILE = _N_INC // _NS
_C = 128
_NFULL = _PER_TILE // _C
_TAIL = _PER_TILE - _NFULL * _C
_NB = _NFULL // 2
_ZROWS = 624
_ZREM = _N_NODES - _NS * _ZROWS


def _mmp_body(x_ref, w_ref, b_ref, o_ref):
    y = (jnp.dot(x_ref[...], w_ref[...], preferred_element_type=jnp.float32)
         + b_ref[...])
    au = lax.bitcast_convert_type(
        y[:, :_DP].astype(jnp.bfloat16), jnp.uint16).astype(jnp.uint32)
    bu = lax.bitcast_convert_type(
        y[:, _DP:].astype(jnp.bfloat16), jnp.uint16).astype(jnp.uint32)
    o_ref[...] = lax.bitcast_convert_type(au | (bu << 16), jnp.float32)


def _transform_packed(x, w, b, br):
    rows = x.shape[0]
    return pl.pallas_call(
        _mmp_body,
        grid=(rows // br,),
        in_specs=[
            pl.BlockSpec((br, _D), lambda i: (i, 0)),
            pl.BlockSpec((_D, _D), lambda i: (0, 0)),
            pl.BlockSpec((1, _D), lambda i: (0, 0)),
        ],
        out_specs=pl.BlockSpec((br, _DP), lambda i: (i, 0)),
        out_shape=jax.ShapeDtypeStruct((rows, _DP), jnp.float32),
    )(x, w, b.reshape(1, _D))


def _mul_body(a_ref, b_ref, o_ref):
    o_ref[...] = a_ref[...] * b_ref[...]


def _combine(a, b):
    br = 2000
    return pl.pallas_call(
        _mul_body,
        grid=(_N_NODES // br,),
        in_specs=[
            pl.BlockSpec((br, _D), lambda i: (i, 0)),
            pl.BlockSpec((br, _D), lambda i: (i, 0)),
        ],
        out_specs=pl.BlockSpec((br, _D), lambda i: (i, 0)),
        out_shape=jax.ShapeDtypeStruct((_N_NODES, _D), jnp.float32),
    )(a, b)


_SC_OUT = (
    jax.ShapeDtypeStruct((_N_NODES, _D), jnp.float32),
    jax.ShapeDtypeStruct((_N_NODES, _D), jnp.float32),
)
_SC_SCRATCH = [
    pltpu.VMEM((_C, _DP), jnp.float32),
    pltpu.VMEM((_C, _DP), jnp.float32),
    pltpu.VMEM((_C, _D), jnp.float32),
    pltpu.VMEM((_C, _D), jnp.float32),
    pltpu.VMEM((_C,), jnp.int32),
    pltpu.VMEM((_C,), jnp.int32),
    pltpu.VMEM((_C,), jnp.int32),
    pltpu.VMEM((_C,), jnp.int32),
    pltpu.VMEM((_C,), jnp.float32),
    pltpu.VMEM((_C,), jnp.float32),
    pltpu.VMEM((_C,), jnp.int32),
    pltpu.VMEM((_C,), jnp.int32),
    pltpu.VMEM((_TAIL,), jnp.int32),
    pltpu.VMEM_SHARED((_N_NODES, _D), jnp.float32),
    pltpu.SemaphoreType.DMA,
    pltpu.SemaphoreType.DMA,
    pltpu.SemaphoreType.DMA,
    pltpu.SemaphoreType.DMA,
    pltpu.SemaphoreType.DMA,
    pltpu.SemaphoreType.DMA,
]


@functools.partial(
    pl.kernel,
    out_type=_SC_OUT,
    mesh=plsc.VectorSubcoreMesh(core_axis_name="c", subcore_axis_name="s"),
    scratch_types=_SC_SCRATCH,
    compiler_params=pltpu.CompilerParams(use_tc_tiling_on_sc=False),
)
def _sc_scatter(tn, th, ns, nr, ncv, hs, hr, hcv, out_msg, out_scale,
                rin0, rin1, rout0, rout1, sidx0, sidx1,
                rpk0, rpk1, cvb0, cvb1, ridx0, ridx1, ridx_t, acc,
                gsem0, gsem1, ssem0, ssem1, psem0, psem1):
    cid = lax.axis_index("c")
    sid = lax.axis_index("s")
    rin = (rin0, rin1)
    rout = (rout0, rout1)
    sidx = (sidx0, sidx1)
    rpk = (rpk0, rpk1)
    cvb = (cvb0, cvb1)
    ridx = (ridx0, ridx1)
    gsem = (gsem0, gsem1)
    ssem = (ssem0, ssem1)
    psem = (psem0, psem1)

    def zrow(i, carry):
        for f in range(8):
            rout0[i, pl.ds(f * 16, 16)] = jnp.zeros((16,), jnp.float32)
        return carry

    lax.fori_loop(0, _C, zrow, 0)
    zb = sid * _ZROWS
    for k in range(4):
        pltpu.sync_copy(rout0, acc.at[pl.ds(zb + k * _C, _C)])
    rem = _ZROWS - 4 * _C
    pltpu.sync_copy(rout0.at[pl.ds(0, rem)], acc.at[pl.ds(zb + 4 * _C, rem)])

    @pl.when(sid == _NS - 1)
    def _():
        pltpu.sync_copy(rout0.at[pl.ds(0, _ZREM)],
                        acc.at[pl.ds(_NS * _ZROWS, _ZREM)])

    plsc.subcore_barrier()

    def stream(t_hbm, s_hbm, r_hbm, c_hbm):
        base = sid * _PER_TILE

        def pload(j, s):
            b = base + j * _C
            pltpu.async_copy(s_hbm.at[pl.ds(b, _C)], sidx[s], psem[s])
            pltpu.async_copy(r_hbm.at[pl.ds(b, _C)], rpk[s], psem[s])
            pltpu.async_copy(c_hbm.at[pl.ds(b, _C)], cvb[s], psem[s])

        def pwait(j, s):
            b = base + j * _C
            pltpu.make_async_copy(s_hbm.at[pl.ds(b, _C)], sidx[s], psem[s]).wait()
            pltpu.make_async_copy(r_hbm.at[pl.ds(b, _C)], rpk[s], psem[s]).wait()
            pltpu.make_async_copy(c_hbm.at[pl.ds(b, _C)], cvb[s], psem[s]).wait()

        def gstart(s):
            pltpu.async_copy(t_hbm.at[sidx[s]], rin[s], gsem[s])

        def gwait(s):
            pltpu.make_async_copy(t_hbm.at[sidx[s]], rin[s], gsem[s]).wait()

        def swait(s):
            pltpu.make_async_copy(rout[s], acc.at[ridx[s]], ssem[s]).wait()

        def scale(s, ngroups):
            def grp(g, carry):
                cv16 = cvb[s][pl.ds(g * 16, 16)]
                for r in range(16):
                    i = g * 16 + r
                    cval = jnp.broadcast_to(cv16[r], (16,))
                    for f in range(4):
                        wi = lax.bitcast_convert_type(
                            rin[s][i, pl.ds(f * 16, 16)], jnp.int32)
                        a = lax.bitcast_convert_type(
                            jnp.left_shift(wi, 16), jnp.float32)
                        b = lax.bitcast_convert_type(
                            jnp.bitwise_and(wi, jnp.int32(-65536)),
                            jnp.float32)
                        rout[s][i, pl.ds(f * 16, 16)] = a * cval
                        rout[s][i, pl.ds(_DP + f * 16, 16)] = b * cval
                return carry

            lax.fori_loop(0, ngroups, grp, 0)

        def copy_ridx(s, n):
            dst = ridx_t if n == _TAIL else ridx[s]
            for g in range(n // 16):
                dst[pl.ds(g * 16, 16)] = rpk[s][pl.ds(g * 16, 16)]

        pload(0, 0)
        pload(1, 1)
        pwait(0, 0)
        gstart(0)

        def body(k, carry):
            for s in range(2):
                j = 2 * k + s
                o = 1 - s

                gwait(s)

                @pl.when(k > 0)
                def _(s=s):
                    swait(s)

                @pl.when(j + 1 < _NFULL)
                def _(s=s, o=o, j=j):
                    pwait(j + 1, o)
                    gstart(o)

                scale(s, _C // 16)
                copy_ridx(s, _C)
                pltpu.async_copy(rout[s], acc.at[ridx[s]], ssem[s], add=True)

                @pl.when(j + 2 < _NFULL)
                def _(s=s, j=j):
                    pload(j + 2, s)

            return carry

        lax.fori_loop(0, _NB, body, 0)

        swait(0)
        swait(1)
        tb = base + _NFULL * _C
        pltpu.sync_copy(s_hbm.at[pl.ds(tb, _TAIL)], sidx0.at[pl.ds(0, _TAIL)])
        pltpu.sync_copy(r_hbm.at[pl.ds(tb, _TAIL)], rpk0.at[pl.ds(0, _TAIL)])
        pltpu.sync_copy(c_hbm.at[pl.ds(tb, _TAIL)], cvb0.at[pl.ds(0, _TAIL)])
        pltpu.async_copy(t_hbm.at[sidx0.at[pl.ds(0, _TAIL)]],
                         rin0.at[pl.ds(0, _TAIL)], gsem0).wait()
        scale(0, _TAIL // 16)
        copy_ridx(0, _TAIL)
        pltpu.sync_copy(rout0.at[pl.ds(0, _TAIL)], acc.at[ridx_t], add=True)

    @pl.when(cid == 0)
    def _():
        stream(tn, ns, nr, ncv)

    @pl.when(cid == 1)
    def _():
        stream(th, hs, hr, hcv)

    plsc.subcore_barrier()
    ob = sid * _ZROWS

    @pl.when(cid == 0)
    def _():
        pltpu.sync_copy(acc.at[pl.ds(ob, _ZROWS)], out_msg.at[pl.ds(ob, _ZROWS)])

        @pl.when(sid == _NS - 1)
        def _():
            pltpu.sync_copy(acc.at[pl.ds(_NS * _ZROWS, _ZREM)],
                            out_msg.at[pl.ds(_NS * _ZROWS, _ZREM)])

    @pl.when(cid == 1)
    def _():
        pltpu.sync_copy(acc.at[pl.ds(ob, _ZROWS)],
                        out_scale.at[pl.ds(ob, _ZROWS)])

        @pl.when(sid == _NS - 1)
        def _():
            pltpu.sync_copy(acc.at[pl.ds(_NS * _ZROWS, _ZREM)],
                            out_scale.at[pl.ds(_NS * _ZROWS, _ZREM)])


def kernel(node_features, hedge_features, node_senders, node_receivers,
           node_convolution, hedge2node_senders, hedge2node_receivers,
           hedge2node_convolution, W_msg, b_msg, W_scale, b_scale):
    tn = _transform_packed(node_features, W_msg, b_msg, 2000)
    th = _transform_packed(hedge_features, W_scale, b_scale, 2000)
    s_msg, s_scale = _sc_scatter(
        tn, th,
        node_senders.astype(jnp.int32),
        node_receivers.astype(jnp.int32),
        node_convolution.reshape(-1),
        hedge2node_senders.astype(jnp.int32),
        hedge2node_receivers.astype(jnp.int32),
        hedge2node_convolution.reshape(-1),
    )
    return _combine(s_msg, s_scale)


# speedup vs baseline: 2.1018x; 2.1018x over previous
import functools

import jax
import jax.numpy as jnp
from jax import lax
from jax.experimental import pallas as pl
from jax.experimental.pallas import tpu as pltpu
from jax.experimental.pallas import tpu_sc as plsc

_N_NODES = 10000
_N_HEDGES = 20000
_N_INC = 320000
_D = 128

_NS = 16
_PER_TILE = _N_INC // _NS
_C = 64
_NFULL = _PER_TILE // _C
_TAIL = _PER_TILE - _NFULL * _C
_NSLOT = 4
_NB = _NFULL // _NSLOT
_ZROWS = 624
_ZREM = _N_NODES - _NS * _ZROWS


def _mm_body(x_ref, w_ref, b_ref, o_ref):
    o_ref[...] = (
        jnp.dot(x_ref[...], w_ref[...], preferred_element_type=jnp.float32)
        + b_ref[...]
    )


def _transform(x, w, b, br):
    rows = x.shape[0]
    return pl.pallas_call(
        _mm_body,
        grid=(rows // br,),
        in_specs=[
            pl.BlockSpec((br, _D), lambda i: (i, 0)),
            pl.BlockSpec((_D, _D), lambda i: (0, 0)),
            pl.BlockSpec((1, _D), lambda i: (0, 0)),
        ],
        out_specs=pl.BlockSpec((br, _D), lambda i: (i, 0)),
        out_shape=jax.ShapeDtypeStruct((rows, _D), jnp.float32),
    )(x, w, b.reshape(1, _D))


def _mul_body(a_ref, b_ref, o_ref):
    o_ref[...] = a_ref[...] * b_ref[...]


def _combine(a, b):
    br = 2000
    return pl.pallas_call(
        _mul_body,
        grid=(_N_NODES // br,),
        in_specs=[
            pl.BlockSpec((br, _D), lambda i: (i, 0)),
            pl.BlockSpec((br, _D), lambda i: (i, 0)),
        ],
        out_specs=pl.BlockSpec((br, _D), lambda i: (i, 0)),
        out_shape=jax.ShapeDtypeStruct((_N_NODES, _D), jnp.float32),
    )(a, b)


_SC_OUT = (
    jax.ShapeDtypeStruct((_N_NODES, _D), jnp.float32),
    jax.ShapeDtypeStruct((_N_NODES, _D), jnp.float32),
)
_SC_SCRATCH = (
    [pltpu.VMEM((_C, _D), jnp.float32)] * _NSLOT
    + [pltpu.VMEM((_C,), jnp.int32)] * _NSLOT
    + [pltpu.VMEM((_C,), jnp.int32)] * _NSLOT
    + [pltpu.VMEM((_C,), jnp.float32)] * _NSLOT
    + [pltpu.VMEM((_C,), jnp.int32)] * _NSLOT
    + [pltpu.VMEM((_TAIL,), jnp.int32)]
    + [pltpu.VMEM_SHARED((_N_NODES, _D), jnp.float32)]
    + [pltpu.SemaphoreType.DMA] * (3 * _NSLOT)
)


@functools.partial(
    pl.kernel,
    out_type=_SC_OUT,
    mesh=plsc.VectorSubcoreMesh(core_axis_name="c", subcore_axis_name="s"),
    scratch_types=_SC_SCRATCH,
)
def _sc_scatter(tn, th, ns, nr, ncv, hs, hr, hcv, out_msg, out_scale,
                rows0, rows1, rows2, rows3, sidx0, sidx1, sidx2, sidx3,
                rpk0, rpk1, rpk2, rpk3, cvb0, cvb1, cvb2, cvb3,
                ridx0, ridx1, ridx2, ridx3, ridx_t, acc,
                gsem0, gsem1, gsem2, gsem3, ssem0, ssem1, ssem2, ssem3,
                psem0, psem1, psem2, psem3):
    cid = lax.axis_index("c")
    sid = lax.axis_index("s")
    rows = (rows0, rows1, rows2, rows3)
    sidx = (sidx0, sidx1, sidx2, sidx3)
    rpk = (rpk0, rpk1, rpk2, rpk3)
    cvb = (cvb0, cvb1, cvb2, cvb3)
    ridx = (ridx0, ridx1, ridx2, ridx3)
    gsem = (gsem0, gsem1, gsem2, gsem3)
    ssem = (ssem0, ssem1, ssem2, ssem3)
    psem = (psem0, psem1, psem2, psem3)

    def zrow(i, carry):
        for f in range(8):
            rows0[i, pl.ds(f * 16, 16)] = jnp.zeros((16,), jnp.float32)
        return carry

    lax.fori_loop(0, _C, zrow, 0)
    zb = sid * _ZROWS
    for k in range(9):
        pltpu.sync_copy(rows0, acc.at[pl.ds(zb + k * _C, _C)])
    rem = _ZROWS - 9 * _C
    pltpu.sync_copy(rows0.at[pl.ds(0, rem)], acc.at[pl.ds(zb + 9 * _C, rem)])

    @pl.when(sid == _NS - 1)
    def _():
        pltpu.sync_copy(rows0.at[pl.ds(0, _ZREM)],
                        acc.at[pl.ds(_NS * _ZROWS, _ZREM)])

    plsc.subcore_barrier()

    def stream(t_hbm, s_hbm, r_hbm, c_hbm):
        base = sid * _PER_TILE

        def pload(j, s):
            b = base + j * _C
            pltpu.async_copy(s_hbm.at[pl.ds(b, _C)], sidx[s], psem[s])
            pltpu.async_copy(r_hbm.at[pl.ds(b, _C)], rpk[s], psem[s])
            pltpu.async_copy(c_hbm.at[pl.ds(b, _C)], cvb[s], psem[s])

        def pwait(j, s):
            b = base + j * _C
            pltpu.make_async_copy(s_hbm.at[pl.ds(b, _C)], sidx[s], psem[s]).wait()
            pltpu.make_async_copy(r_hbm.at[pl.ds(b, _C)], rpk[s], psem[s]).wait()
            pltpu.make_async_copy(c_hbm.at[pl.ds(b, _C)], cvb[s], psem[s]).wait()

        def gstart(s):
            pltpu.async_copy(t_hbm.at[sidx[s]], rows[s], gsem[s])

        def gwait(s):
            pltpu.make_async_copy(t_hbm.at[sidx[s]], rows[s], gsem[s]).wait()

        def swait(s):
            pltpu.make_async_copy(rows[s], acc.at[ridx[s]], ssem[s]).wait()

        def scale(s, ngroups):
            def grp(g, carry):
                cv16 = cvb[s][pl.ds(g * 16, 16)]
                for r in range(16):
                    i = g * 16 + r
                    cval = jnp.broadcast_to(cv16[r], (16,))
                    for f in range(8):
                        sl = pl.ds(f * 16, 16)
                        rows[s][i, sl] = rows[s][i, sl] * cval
                return carry

            lax.fori_loop(0, ngroups, grp, 0)

        def copy_ridx(s, n):
            dst = ridx_t if n == _TAIL else ridx[s]
            for g in range(n // 16):
                dst[pl.ds(g * 16, 16)] = rpk[s][pl.ds(g * 16, 16)]

        for s in range(_NSLOT):
            pload(s, s)
        pwait(0, 0)
        gstart(0)
        pwait(1, 1)
        gstart(1)

        def body(k, carry):
            for s in range(_NSLOT):
                j = _NSLOT * k + s
                nx2 = (s + 2) % _NSLOT

                @pl.when(j + 2 < _NFULL)
                def _(s=s, nx2=nx2, j=j):
                    pwait(j + 2, nx2)
                    if s >= 2:
                        swait(nx2)
                    else:
                        @pl.when(k > 0)
                        def _():
                            swait(nx2)

                    gstart(nx2)

                gwait(s)
                scale(s, _C // 16)
                copy_ridx(s, _C)
                pltpu.async_copy(rows[s], acc.at[ridx[s]], ssem[s], add=True)

                @pl.when(j + _NSLOT < _NFULL)
                def _(s=s, j=j):
                    pload(j + _NSLOT, s)

            return carry

        lax.fori_loop(0, _NB, body, 0)

        for s in range(_NSLOT):
            swait(s)
        tb = base + _NFULL * _C
        pltpu.sync_copy(s_hbm.at[pl.ds(tb, _TAIL)], sidx0.at[pl.ds(0, _TAIL)])
        pltpu.sync_copy(r_hbm.at[pl.ds(tb, _TAIL)], rpk0.at[pl.ds(0, _TAIL)])
        pltpu.sync_copy(c_hbm.at[pl.ds(tb, _TAIL)], cvb0.at[pl.ds(0, _TAIL)])
        pltpu.async_copy(t_hbm.at[sidx0.at[pl.ds(0, _TAIL)]],
                         rows0.at[pl.ds(0, _TAIL)], gsem0).wait()
        scale(0, _TAIL // 16)
        copy_ridx(0, _TAIL)
        pltpu.sync_copy(rows0.at[pl.ds(0, _TAIL)], acc.at[ridx_t], add=True)

    @pl.when(cid == 0)
    def _():
        stream(tn, ns, nr, ncv)

    @pl.when(cid == 1)
    def _():
        stream(th, hs, hr, hcv)

    plsc.subcore_barrier()
    ob = sid * _ZROWS

    @pl.when(cid == 0)
    def _():
        pltpu.sync_copy(acc.at[pl.ds(ob, _ZROWS)], out_msg.at[pl.ds(ob, _ZROWS)])

        @pl.when(sid == _NS - 1)
        def _():
            pltpu.sync_copy(acc.at[pl.ds(_NS * _ZROWS, _ZREM)],
                            out_msg.at[pl.ds(_NS * _ZROWS, _ZREM)])

    @pl.when(cid == 1)
    def _():
        pltpu.sync_copy(acc.at[pl.ds(ob, _ZROWS)],
                        out_scale.at[pl.ds(ob, _ZROWS)])

        @pl.when(sid == _NS - 1)
        def _():
            pltpu.sync_copy(acc.at[pl.ds(_NS * _ZROWS, _ZREM)],
                            out_scale.at[pl.ds(_NS * _ZROWS, _ZREM)])


def kernel(node_features, hedge_features, node_senders, node_receivers,
           node_convolution, hedge2node_senders, hedge2node_receivers,
           hedge2node_convolution, W_msg, b_msg, W_scale, b_scale):
    tn = _transform(node_features, W_msg, b_msg, 2000)
    th = _transform(hedge_features, W_scale, b_scale, 2000)
    s_msg, s_scale = _sc_scatter(
        tn, th,
        node_senders.astype(jnp.int32),
        node_receivers.astype(jnp.int32),
        node_convolution.reshape(-1),
        hedge2node_senders.astype(jnp.int32),
        hedge2node_receivers.astype(jnp.int32),
        hedge2node_convolution.reshape(-1),
    )
    return _combine(s_msg, s_scale)
